# baseline (speedup 1.0000x reference)
import functools

import jax
import jax.numpy as jnp
from jax import lax
from jax.experimental import pallas as pl
from jax.experimental.pallas import tpu as pltpu
from jax.experimental.pallas import tpu_sc as plsc

N = 10000
D = 128
H = 64
E = 320000

NC = 2
NS = 16
NW = NC * NS

B = 128
K = 80
NB = 2
NG = K // NB
EPT = K * B
E_PAD = NW * EPT
N_ACC = 10240
RPT = N_ACC // NS
ZB = 128


def _seg_pass_body(with_counts, *refs):
    if with_counts:
        (y_h, src_h, dst_h, out_h, cnt_h,
         src_v, dst_v, rows_v, zero_v, ones_v, zcnt_v, acc_sh, cnt_sh,
         semg) = refs
    else:
        (y_h, src_h, dst_h, out_h,
         src_v, dst_v, rows_v, zero_v, acc_sh, semg) = refs

    cid = lax.axis_index("c")
    sid = lax.axis_index("s")
    w = cid * NS + sid

    def fill_zero(i, _):
        zero_v[i // 4, pl.ds((i % 4) * 16, 16)] = jnp.zeros((16,), jnp.float32)
        return 0
    lax.fori_loop(0, ZB * 4, fill_zero, 0)
    if with_counts:
        def fill_misc(i, _):
            ones_v[pl.ds(i * 16, 16)] = jnp.ones((16,), jnp.float32)
            return 0
        lax.fori_loop(0, B // 16, fill_misc, 0)
        def fill_zcnt(i, _):
            zcnt_v[pl.ds(i * 16, 16)] = jnp.zeros((16,), jnp.float32)
            return 0
        lax.fori_loop(0, RPT // 16, fill_zcnt, 0)

    for b in range(RPT // ZB):
        pltpu.sync_copy(zero_v, acc_sh.at[pl.ds(sid * RPT + b * ZB, ZB)])
    if with_counts:
        pltpu.sync_copy(zcnt_v, cnt_sh.at[pl.ds(sid * RPT, RPT)])
    plsc.subcore_barrier()

    pltpu.sync_copy(src_h.at[w], src_v)
    pltpu.sync_copy(dst_h.at[w], dst_v)

    pltpu.async_copy(y_h.at[src_v.at[0]], rows_v.at[0], semg.at[0])

    def group(g, _):
        for b in range(NB):
            j = g * NB + b
            pltpu.make_async_copy(y_h.at[pl.ds(0, B)], rows_v.at[b],
                                  semg.at[b]).wait()

            @pl.when(j < K - 1)
            def _():
                b2 = (b + 1) % NB
                pltpu.async_copy(y_h.at[src_v.at[j + 1]], rows_v.at[b2],
                                 semg.at[b2])
            pltpu.sync_copy(rows_v.at[b], acc_sh.at[dst_v.at[j]], add=True)
            if with_counts:
                pltpu.sync_copy(ones_v, cnt_sh.at[dst_v.at[j]], add=True)
        return 0
    lax.fori_loop(0, NG, group, 0)

    plsc.subcore_barrier()

    pltpu.sync_copy(acc_sh.at[pl.ds(sid * RPT, RPT)],
                    out_h.at[cid, pl.ds(sid * RPT, RPT)])
    if with_counts:
        pltpu.sync_copy(cnt_sh.at[pl.ds(sid * RPT, RPT)],
                        cnt_h.at[cid, pl.ds(sid * RPT, RPT)])


def _make_seg_pass(with_counts):
    mesh = plsc.VectorSubcoreMesh(core_axis_name="c", subcore_axis_name="s")
    out_type = [jax.ShapeDtypeStruct((NC, N_ACC, H), jnp.float32)]
    scratch = [
        pltpu.VMEM((K, B), jnp.int32),
        pltpu.VMEM((K, B), jnp.int32),
        pltpu.VMEM((NB, B, H), jnp.float32),
        pltpu.VMEM((ZB, H), jnp.float32),
    ]
    if with_counts:
        out_type.append(jax.ShapeDtypeStruct((NC, N_ACC), jnp.float32))
        scratch += [
            pltpu.VMEM((B,), jnp.float32),
            pltpu.VMEM((RPT,), jnp.float32),
        ]
    scratch.append(pltpu.VMEM_SHARED((N_ACC, H), jnp.float32))
    if with_counts:
        scratch.append(pltpu.VMEM_SHARED((N_ACC,), jnp.float32))
    scratch.append(pltpu.SemaphoreType.DMA((NB,)))
    return pl.kernel(
        functools.partial(_seg_pass_body, with_counts),
        out_type=tuple(out_type),
        mesh=mesh,
        scratch_types=tuple(scratch),
        compiler_params=pltpu.CompilerParams(use_tc_tiling_on_sc=False),
    )


_seg_pass_cnt = _make_seg_pass(True)
_seg_pass = _make_seg_pass(False)


def _bn_elu(z, g, be):
    m = jnp.mean(z, axis=0)
    v = jnp.mean((z - m) ** 2, axis=0)
    zn = (z - m) * lax.rsqrt(v + 1e-5) * g + be
    return jnp.where(zn > 0, zn, jnp.exp(zn) - 1.0)


def _mm(a, w):
    return jnp.dot(a, w, preferred_element_type=jnp.float32)


def _stage_a(x_ref, wl_ref, wr_ref, y_ref, xr_ref):
    x = x_ref[...]
    y_ref[...] = _mm(x, wl_ref[...])
    xr_ref[...] = _mm(x, wr_ref[...])


def _sum_parts(s_ref, cnt_ref):
    s = (s_ref[0] + s_ref[1])[:N, :]
    cnt = jnp.maximum((cnt_ref[0] + cnt_ref[1])[:N], 1.0)
    return s / cnt[:, None]


def _stage_b(s_ref, cnt_ref, xr_ref, bl_ref, g_ref, be_ref,
             wl2_ref, wr2_ref, wlin_ref, y_ref, xr2_ref, xl_ref):
    z = _sum_parts(s_ref, cnt_ref) + bl_ref[...] + xr_ref[...]
    x1 = _bn_elu(z, g_ref[...], be_ref[...])
    y_ref[...] = _mm(x1, wl2_ref[...])
    xr2_ref[...] = _mm(x1, wr2_ref[...])
    xl_ref[...] = _mm(x1, wlin_ref[...])


def _stage_c(s_ref, cnt_ref, xr_ref, bl_ref, g_ref, be_ref, xl_ref, blin_ref,
             wl2_ref, wr2_ref, wlin_ref, y_ref, xr2_ref, xl2_ref):
    z = _sum_parts(s_ref, cnt_ref) + bl_ref[...] + xr_ref[...]
    x2 = _bn_elu(z, g_ref[...], be_ref[...]) + xl_ref[...] + blin_ref[...]
    y_ref[...] = _mm(x2, wl2_ref[...])
    xr2_ref[...] = _mm(x2, wr2_ref[...])
    xl2_ref[...] = _mm(x2, wlin_ref[...])


def _stage_d(s_ref, cnt_ref, xr_ref, bl_ref, g_ref, be_ref, xl_ref, blin_ref,
             wl4_ref, y_ref, x3_ref):
    z = _sum_parts(s_ref, cnt_ref) + bl_ref[...] + xr_ref[...]
    x3 = _bn_elu(z, g_ref[...], be_ref[...]) + xl_ref[...] + blin_ref[...]
    x3_ref[...] = x3
    y4 = _mm(x3, wl4_ref[...])
    y_ref[...] = jnp.broadcast_to(y4, (N, H))


def _stage_e(s_ref, cnt_ref, x3_ref, wr4_ref, b4_ref, out_ref):
    s = (s_ref[0] + s_ref[1])[:N, :1]
    cnt = jnp.maximum((cnt_ref[0] + cnt_ref[1])[:N], 1.0)
    agg = s / cnt[:, None]
    out_ref[...] = agg + b4_ref[...] + _mm(x3_ref[...], wr4_ref[...])


def _tc(body, out_shapes):
    return pl.pallas_call(body, out_shape=out_shapes)


_f32 = jnp.float32
_NH = jax.ShapeDtypeStruct((N, H), _f32)


def kernel(x, edge_index, W1l, b1l, W1r, W2l, b2l, W2r, W3l, b3l, W3r,
           W4l, b4l, W4r, g1, be1, g2, be2, g3, be3, Wlin, blin):
    src = edge_index[0]
    dst = edge_index[1]
    pad = E_PAD - E
    src_p = jnp.concatenate([src, jnp.zeros((pad,), jnp.int32)])
    dst_p = jnp.concatenate([dst, jnp.full((pad,), N, jnp.int32)])
    srcs = src_p.reshape(NW, K, B)
    dsts = dst_p.reshape(NW, K, B)

    y1, xr1 = _tc(_stage_a, (_NH, _NH))(x, W1l, W1r)
    s1, cnt = _seg_pass_cnt(y1, srcs, dsts)
    y2, xr2, xl1 = _tc(_stage_b, (_NH, _NH, _NH))(
        s1, cnt, xr1, b1l, g1, be1, W2l, W2r, Wlin)
    (s2,) = _seg_pass(y2, srcs, dsts)
    y3, xr3, xl2 = _tc(_stage_c, (_NH, _NH, _NH))(
        s2, cnt, xr2, b2l, g2, be2, xl1, blin, W3l, W3r, Wlin)
    (s3,) = _seg_pass(y3, srcs, dsts)
    y4, x3 = _tc(_stage_d, (_NH, _NH))(
        s3, cnt, xr3, b3l, g3, be3, xl2, blin, W4l)
    (s4,) = _seg_pass(y4, srcs, dsts)
    out = _tc(_stage_e, jax.ShapeDtypeStruct((N, 1), _f32))(
        s4, cnt, x3, W4r, b4l)
    return jnp.squeeze(out, axis=-1)


# speedup vs baseline: 2.0570x; 2.0570x over previous
import functools

import jax
import jax.numpy as jnp
from jax import lax
from jax.experimental import pallas as pl
from jax.experimental.pallas import tpu as pltpu
from jax.experimental.pallas import tpu_sc as plsc

N = 10000
D = 128
H = 64
E = 320000

NC = 2
NS = 16
NW = NC * NS

B = 128
K = 79
EPT = K * B
E_PAD = NW * EPT
N_ACC = 10240
RPT = N_ACC // NS
ZB = 128


def _seg_pass_body(with_counts, *refs):
    if with_counts:
        (y_h, src_h, dst_h, out_h, cnt_h,
         src_v, dst_v, rows_v, zero_v, ones_v, zcnt_v, acc_sh, cnt_sh,
         y_sh) = refs
    else:
        (y_h, src_h, dst_h, out_h,
         src_v, dst_v, rows_v, zero_v, acc_sh, y_sh) = refs

    cid = lax.axis_index("c")
    sid = lax.axis_index("s")
    w = cid * NS + sid

    def fill_zero(i, _):
        zero_v[i // 4, pl.ds((i % 4) * 16, 16)] = jnp.zeros((16,), jnp.float32)
        return 0
    lax.fori_loop(0, ZB * 4, fill_zero, 0)
    if with_counts:
        def fill_misc(i, _):
            ones_v[pl.ds(i * 16, 16)] = jnp.ones((16,), jnp.float32)
            return 0
        lax.fori_loop(0, B // 16, fill_misc, 0)
        def fill_zcnt(i, _):
            zcnt_v[pl.ds(i * 16, 16)] = jnp.zeros((16,), jnp.float32)
            return 0
        lax.fori_loop(0, RPT // 16, fill_zcnt, 0)

    for b in range(RPT // ZB):
        pltpu.sync_copy(zero_v, acc_sh.at[pl.ds(sid * RPT + b * ZB, ZB)])
    if with_counts:
        pltpu.sync_copy(zcnt_v, cnt_sh.at[pl.ds(sid * RPT, RPT)])
    plsc.subcore_barrier()

    pltpu.sync_copy(src_h.at[w], src_v)
    pltpu.sync_copy(dst_h.at[w], dst_v)

    pltpu.sync_copy(y_h.at[pl.ds(sid * (N // NS), N // NS)],
                    y_sh.at[pl.ds(sid * (N // NS), N // NS)])
    plsc.subcore_barrier()

    def blk(j, _):
        pltpu.sync_copy(y_sh.at[src_v.at[j]], rows_v)
        pltpu.sync_copy(rows_v, acc_sh.at[dst_v.at[j]], add=True)
        if with_counts:
            pltpu.sync_copy(ones_v, cnt_sh.at[dst_v.at[j]], add=True)
        return 0
    lax.fori_loop(0, K, blk, 0)

    plsc.subcore_barrier()

    pltpu.sync_copy(acc_sh.at[pl.ds(sid * RPT, RPT)],
                    out_h.at[cid, pl.ds(sid * RPT, RPT)])
    if with_counts:
        pltpu.sync_copy(cnt_sh.at[pl.ds(sid * RPT, RPT)],
                        cnt_h.at[cid, pl.ds(sid * RPT, RPT)])


def _make_seg_pass(with_counts):
    mesh = plsc.VectorSubcoreMesh(core_axis_name="c", subcore_axis_name="s")
    out_type = [jax.ShapeDtypeStruct((NC, N_ACC, H), jnp.float32)]
    scratch = [
        pltpu.VMEM((K, B), jnp.int32),
        pltpu.VMEM((K, B), jnp.int32),
        pltpu.VMEM((B, H), jnp.float32),
        pltpu.VMEM((ZB, H), jnp.float32),
    ]
    if with_counts:
        out_type.append(jax.ShapeDtypeStruct((NC, N_ACC), jnp.float32))
        scratch += [
            pltpu.VMEM((B,), jnp.float32),
            pltpu.VMEM((RPT,), jnp.float32),
        ]
    scratch.append(pltpu.VMEM_SHARED((N_ACC, H), jnp.float32))
    if with_counts:
        scratch.append(pltpu.VMEM_SHARED((N_ACC,), jnp.float32))
    scratch.append(pltpu.VMEM_SHARED((N, H), jnp.float32))
    return pl.kernel(
        functools.partial(_seg_pass_body, with_counts),
        out_type=tuple(out_type),
        mesh=mesh,
        scratch_types=tuple(scratch),
        compiler_params=pltpu.CompilerParams(use_tc_tiling_on_sc=False),
    )


_seg_pass_cnt = _make_seg_pass(True)
_seg_pass = _make_seg_pass(False)


def _bn_elu(z, g, be):
    m = jnp.mean(z, axis=0)
    v = jnp.mean((z - m) ** 2, axis=0)
    zn = (z - m) * lax.rsqrt(v + 1e-5) * g + be
    return jnp.where(zn > 0, zn, jnp.exp(zn) - 1.0)


def _mm(a, w):
    return jnp.dot(a, w, preferred_element_type=jnp.float32)


def _stage_a(x_ref, wl_ref, wr_ref, y_ref, xr_ref):
    x = x_ref[...]
    y_ref[...] = _mm(x, wl_ref[...])
    xr_ref[...] = _mm(x, wr_ref[...])


def _sum_parts(s_ref, cnt_ref):
    s = (s_ref[0] + s_ref[1])[:N, :]
    cnt = jnp.maximum((cnt_ref[0] + cnt_ref[1])[:N], 1.0)
    return s / cnt[:, None]


def _stage_b(s_ref, cnt_ref, xr_ref, bl_ref, g_ref, be_ref,
             wl2_ref, wr2_ref, wlin_ref, y_ref, xr2_ref, xl_ref):
    z = _sum_parts(s_ref, cnt_ref) + bl_ref[...] + xr_ref[...]
    x1 = _bn_elu(z, g_ref[...], be_ref[...])
    y_ref[...] = _mm(x1, wl2_ref[...])
    xr2_ref[...] = _mm(x1, wr2_ref[...])
    xl_ref[...] = _mm(x1, wlin_ref[...])


def _stage_c(s_ref, cnt_ref, xr_ref, bl_ref, g_ref, be_ref, xl_ref, blin_ref,
             wl2_ref, wr2_ref, wlin_ref, y_ref, xr2_ref, xl2_ref):
    z = _sum_parts(s_ref, cnt_ref) + bl_ref[...] + xr_ref[...]
    x2 = _bn_elu(z, g_ref[...], be_ref[...]) + xl_ref[...] + blin_ref[...]
    y_ref[...] = _mm(x2, wl2_ref[...])
    xr2_ref[...] = _mm(x2, wr2_ref[...])
    xl2_ref[...] = _mm(x2, wlin_ref[...])


def _stage_d(s_ref, cnt_ref, xr_ref, bl_ref, g_ref, be_ref, xl_ref, blin_ref,
             wl4_ref, y_ref, x3_ref):
    z = _sum_parts(s_ref, cnt_ref) + bl_ref[...] + xr_ref[...]
    x3 = _bn_elu(z, g_ref[...], be_ref[...]) + xl_ref[...] + blin_ref[...]
    x3_ref[...] = x3
    y4 = _mm(x3, wl4_ref[...])
    y_ref[...] = jnp.broadcast_to(y4, (N, H))


def _stage_e(s_ref, cnt_ref, x3_ref, wr4_ref, b4_ref, out_ref):
    s = (s_ref[0] + s_ref[1])[:N, :1]
    cnt = jnp.maximum((cnt_ref[0] + cnt_ref[1])[:N], 1.0)
    agg = s / cnt[:, None]
    out_ref[...] = agg + b4_ref[...] + _mm(x3_ref[...], wr4_ref[...])


def _tc(body, out_shapes):
    return pl.pallas_call(body, out_shape=out_shapes)


_f32 = jnp.float32
_NH = jax.ShapeDtypeStruct((N, H), _f32)


def kernel(x, edge_index, W1l, b1l, W1r, W2l, b2l, W2r, W3l, b3l, W3r,
           W4l, b4l, W4r, g1, be1, g2, be2, g3, be3, Wlin, blin):
    src = edge_index[0]
    dst = edge_index[1]
    pad = E_PAD - E
    src_p = jnp.concatenate([src, jnp.zeros((pad,), jnp.int32)])
    dst_p = jnp.concatenate([dst, jnp.full((pad,), N, jnp.int32)])
    srcs = src_p.reshape(NW, K, B)
    dsts = dst_p.reshape(NW, K, B)

    y1, xr1 = _tc(_stage_a, (_NH, _NH))(x, W1l, W1r)
    s1, cnt = _seg_pass_cnt(y1, srcs, dsts)
    y2, xr2, xl1 = _tc(_stage_b, (_NH, _NH, _NH))(
        s1, cnt, xr1, b1l, g1, be1, W2l, W2r, Wlin)
    (s2,) = _seg_pass(y2, srcs, dsts)
    y3, xr3, xl2 = _tc(_stage_c, (_NH, _NH, _NH))(
        s2, cnt, xr2, b2l, g2, be2, xl1, blin, W3l, W3r, Wlin)
    (s3,) = _seg_pass(y3, srcs, dsts)
    y4, x3 = _tc(_stage_d, (_NH, _NH))(
        s3, cnt, xr3, b3l, g3, be3, xl2, blin, W4l)
    (s4,) = _seg_pass(y4, srcs, dsts)
    out = _tc(_stage_e, jax.ShapeDtypeStruct((N, 1), _f32))(
        s4, cnt, x3, W4r, b4l)
    return jnp.squeeze(out, axis=-1)


# speedup vs baseline: 2.4334x; 1.1830x over previous
import functools

import jax
import jax.numpy as jnp
from jax import lax
from jax.experimental import pallas as pl
from jax.experimental.pallas import tpu as pltpu
from jax.experimental.pallas import tpu_sc as plsc

N = 10000
D = 128
H = 64
E = 320000

NC = 2
NS = 16
NW = NC * NS

B = 128
K = 79
EPT = K * B
E_PAD = NW * EPT
N_ACC = 10240
RPT = N_ACC // NS
ZB = 128


def _seg_pass_body(with_counts, *refs):
    if with_counts:
        (y_h, src_h, dst_h, out_h, cnt_h,
         src_v, dst_v, rows_v, zero_v, ones_v, zcnt_v, acc_sh, cnt_sh,
         y_sh) = refs
    else:
        (y_h, src_h, dst_h, out_h,
         src_v, dst_v, rows_v, zero_v, acc_sh, y_sh) = refs

    cid = lax.axis_index("c")
    sid = lax.axis_index("s")
    w = cid * NS + sid

    def fill_zero(i, _):
        zero_v[i // 4, pl.ds((i % 4) * 16, 16)] = jnp.zeros((16,), jnp.float32)
        return 0
    lax.fori_loop(0, ZB * 4, fill_zero, 0)
    if with_counts:
        def fill_misc(i, _):
            ones_v[pl.ds(i * 16, 16)] = jnp.ones((16,), jnp.float32)
            return 0
        lax.fori_loop(0, B // 16, fill_misc, 0)
        def fill_zcnt(i, _):
            zcnt_v[pl.ds(i * 16, 16)] = jnp.zeros((16,), jnp.float32)
            return 0
        lax.fori_loop(0, RPT // 16, fill_zcnt, 0)

    for b in range(RPT // ZB):
        pltpu.sync_copy(zero_v, acc_sh.at[pl.ds(sid * RPT + b * ZB, ZB)])
    if with_counts:
        pltpu.sync_copy(zcnt_v, cnt_sh.at[pl.ds(sid * RPT, RPT)])
    plsc.subcore_barrier()

    pltpu.sync_copy(src_h.at[w], src_v)
    pltpu.sync_copy(dst_h.at[w], dst_v)

    pltpu.sync_copy(y_h.at[pl.ds(sid * (N // NS), N // NS)],
                    y_sh.at[pl.ds(sid * (N // NS), N // NS)])
    plsc.subcore_barrier()

    def blk(j, _):
        pltpu.sync_copy(y_sh.at[src_v.at[j]], rows_v)
        pltpu.sync_copy(rows_v, acc_sh.at[dst_v.at[j]], add=True)
        if with_counts:
            pltpu.sync_copy(ones_v, cnt_sh.at[dst_v.at[j]], add=True)
        return 0
    lax.fori_loop(0, K, blk, 0)

    plsc.subcore_barrier()

    pltpu.sync_copy(acc_sh.at[pl.ds(sid * RPT, RPT)],
                    out_h.at[cid, pl.ds(sid * RPT, RPT)])
    if with_counts:
        pltpu.sync_copy(cnt_sh.at[pl.ds(sid * RPT, RPT)],
                        cnt_h.at[cid, pl.ds(sid * RPT, RPT)])


def _make_seg_pass(with_counts):
    mesh = plsc.VectorSubcoreMesh(core_axis_name="c", subcore_axis_name="s")
    out_type = [jax.ShapeDtypeStruct((NC, N_ACC, H), jnp.float32)]
    scratch = [
        pltpu.VMEM((K, B), jnp.int32),
        pltpu.VMEM((K, B), jnp.int32),
        pltpu.VMEM((B, H), jnp.float32),
        pltpu.VMEM((ZB, H), jnp.float32),
    ]
    if with_counts:
        out_type.append(jax.ShapeDtypeStruct((NC, N_ACC), jnp.float32))
        scratch += [
            pltpu.VMEM((B,), jnp.float32),
            pltpu.VMEM((RPT,), jnp.float32),
        ]
    scratch.append(pltpu.VMEM_SHARED((N_ACC, H), jnp.float32))
    if with_counts:
        scratch.append(pltpu.VMEM_SHARED((N_ACC,), jnp.float32))
    scratch.append(pltpu.VMEM_SHARED((N, H), jnp.float32))
    return pl.kernel(
        functools.partial(_seg_pass_body, with_counts),
        out_type=tuple(out_type),
        mesh=mesh,
        scratch_types=tuple(scratch),
        compiler_params=pltpu.CompilerParams(use_tc_tiling_on_sc=False),
    )


_seg_pass_cnt = _make_seg_pass(True)
_seg_pass = _make_seg_pass(False)


G16 = EPT // 16


def _seg_scalar_body(y_h, src_h, dst_h, out_h, y_v, src_v, dst_v, acc_v):
    cid = lax.axis_index("c")
    sid = lax.axis_index("s")
    w = cid * NS + sid
    pltpu.sync_copy(y_h, y_v)
    pltpu.sync_copy(src_h.at[w], src_v)
    pltpu.sync_copy(dst_h.at[w], dst_v)

    def z(i, _):
        acc_v[pl.ds(i * 16, 16)] = jnp.zeros((16,), jnp.float32)
        return 0
    lax.fori_loop(0, N_ACC // 16, z, 0)

    def it(i, _):
        vals = plsc.load_gather(y_v, [src_v[i, :]])
        plsc.addupdate_scatter(acc_v, [dst_v[i, :]], vals)
        return 0
    lax.fori_loop(0, G16, it, 0)
    pltpu.sync_copy(acc_v, out_h.at[w])


_seg_scalar = pl.kernel(
    _seg_scalar_body,
    out_type=jax.ShapeDtypeStruct((NW, N_ACC), jnp.float32),
    mesh=plsc.VectorSubcoreMesh(core_axis_name="c", subcore_axis_name="s"),
    scratch_types=(
        pltpu.VMEM((N,), jnp.float32),
        pltpu.VMEM((G16, 16), jnp.int32),
        pltpu.VMEM((G16, 16), jnp.int32),
        pltpu.VMEM((N_ACC,), jnp.float32),
    ),
    compiler_params=pltpu.CompilerParams(use_tc_tiling_on_sc=False,
                                         needs_layout_passes=False),
)


def _bn_elu(z, g, be):
    m = jnp.mean(z, axis=0)
    v = jnp.mean((z - m) ** 2, axis=0)
    zn = (z - m) * lax.rsqrt(v + 1e-5) * g + be
    return jnp.where(zn > 0, zn, jnp.exp(zn) - 1.0)


def _mm(a, w):
    return jnp.dot(a, w, preferred_element_type=jnp.float32)


def _stage_a(x_ref, wl_ref, wr_ref, y_ref, xr_ref):
    x = x_ref[...]
    y_ref[...] = _mm(x, wl_ref[...])
    xr_ref[...] = _mm(x, wr_ref[...])


def _sum_parts(s_ref, cnt_ref):
    s = (s_ref[0] + s_ref[1])[:N, :]
    cnt = jnp.maximum((cnt_ref[0] + cnt_ref[1])[:N], 1.0)
    return s / cnt[:, None]


def _stage_b(s_ref, cnt_ref, xr_ref, bl_ref, g_ref, be_ref,
             wl2_ref, wr2_ref, wlin_ref, y_ref, xr2_ref, xl_ref):
    z = _sum_parts(s_ref, cnt_ref) + bl_ref[...] + xr_ref[...]
    x1 = _bn_elu(z, g_ref[...], be_ref[...])
    y_ref[...] = _mm(x1, wl2_ref[...])
    xr2_ref[...] = _mm(x1, wr2_ref[...])
    xl_ref[...] = _mm(x1, wlin_ref[...])


def _stage_c(s_ref, cnt_ref, xr_ref, bl_ref, g_ref, be_ref, xl_ref, blin_ref,
             wl2_ref, wr2_ref, wlin_ref, y_ref, xr2_ref, xl2_ref):
    z = _sum_parts(s_ref, cnt_ref) + bl_ref[...] + xr_ref[...]
    x2 = _bn_elu(z, g_ref[...], be_ref[...]) + xl_ref[...] + blin_ref[...]
    y_ref[...] = _mm(x2, wl2_ref[...])
    xr2_ref[...] = _mm(x2, wr2_ref[...])
    xl2_ref[...] = _mm(x2, wlin_ref[...])


def _stage_d(s_ref, cnt_ref, xr_ref, bl_ref, g_ref, be_ref, xl_ref, blin_ref,
             wl4_ref, y_ref, x3_ref):
    z = _sum_parts(s_ref, cnt_ref) + bl_ref[...] + xr_ref[...]
    x3 = _bn_elu(z, g_ref[...], be_ref[...]) + xl_ref[...] + blin_ref[...]
    x3_ref[...] = x3
    y_ref[...] = _mm(x3, wl4_ref[...])


def _stage_e(s4t_ref, cnt_ref, x3_ref, wr4_ref, b4_ref, out_ref):
    s = jnp.sum(s4t_ref[...], axis=1, keepdims=True)[:N]
    cnt = jnp.maximum((cnt_ref[0] + cnt_ref[1])[:N], 1.0)
    agg = s / cnt[:, None]
    out_ref[...] = agg + b4_ref[...] + _mm(x3_ref[...], wr4_ref[...])


def _tc(body, out_shapes):
    return pl.pallas_call(body, out_shape=out_shapes)


_f32 = jnp.float32
_NH = jax.ShapeDtypeStruct((N, H), _f32)


def kernel(x, edge_index, W1l, b1l, W1r, W2l, b2l, W2r, W3l, b3l, W3r,
           W4l, b4l, W4r, g1, be1, g2, be2, g3, be3, Wlin, blin):
    src = edge_index[0]
    dst = edge_index[1]
    pad = E_PAD - E
    src_p = jnp.concatenate([src, jnp.zeros((pad,), jnp.int32)])
    dst_p = jnp.concatenate([dst, jnp.full((pad,), N, jnp.int32)])
    srcs = src_p.reshape(NW, K, B)
    dsts = dst_p.reshape(NW, K, B)
    src4 = src_p.reshape(NW, G16, 16)
    dst4 = dst_p.reshape(NW, G16, 16)

    y1, xr1 = _tc(_stage_a, (_NH, _NH))(x, W1l, W1r)
    s1, cnt = _seg_pass_cnt(y1, srcs, dsts)
    y2, xr2, xl1 = _tc(_stage_b, (_NH, _NH, _NH))(
        s1, cnt, xr1, b1l, g1, be1, W2l, W2r, Wlin)
    (s2,) = _seg_pass(y2, srcs, dsts)
    y3, xr3, xl2 = _tc(_stage_c, (_NH, _NH, _NH))(
        s2, cnt, xr2, b2l, g2, be2, xl1, blin, W3l, W3r, Wlin)
    (s3,) = _seg_pass(y3, srcs, dsts)
    y4, x3 = _tc(_stage_d, (jax.ShapeDtypeStruct((N, 1), _f32), _NH))(
        s3, cnt, xr3, b3l, g3, be3, xl2, blin, W4l)
    s4 = _seg_scalar(y4.reshape(N), src4, dst4)
    out = _tc(_stage_e, jax.ShapeDtypeStruct((N, 1), _f32))(
        s4.T, cnt, x3, W4r, b4l)
    return jnp.squeeze(out, axis=-1)
